# baseline (speedup 1.0000x reference)
import jax
import jax.numpy as jnp
from jax import lax
from jax.experimental import pallas as pl
from jax.experimental.pallas import tpu as pltpu
from jax.experimental.pallas import tpu_sc as plsc

B = 16384
D = 64
HIDDEN = 128
N_USERS = 1000000
N_MOVIES = 100000
NC = 2
NS = 16
NW = NC * NS
BPW = B // NW
CH = 128
NCH = BPW // CH
PCOL = 8192
PROW = PCOL // 2


def _pack_body(t_ref, o_ref):
    v = t_ref[...]
    eye = (lax.broadcasted_iota(jnp.int32, (D, D), 0) ==
           lax.broadcasted_iota(jnp.int32, (D, D), 1)).astype(jnp.float32)
    dn = (((0,), (0,)), ((), ()))
    top = lax.dot_general(v[:, 0:PROW], eye, dn,
                          preferred_element_type=jnp.float32,
                          precision=lax.Precision.HIGHEST)
    bot = lax.dot_general(v[:, PROW:PCOL], eye, dn,
                          preferred_element_type=jnp.float32,
                          precision=lax.Precision.HIGHEST)
    o_ref[...] = jnp.concatenate([top, bot], axis=1)


def _make_pack(v_rows):
    grid = (v_rows + PCOL - 1) // PCOL
    return pl.pallas_call(
        _pack_body,
        grid=(grid,),
        in_specs=[pl.BlockSpec((D, PCOL), lambda i: (0, i))],
        out_specs=pl.BlockSpec((PROW, 2 * D), lambda i: (i, 0)),
        out_shape=jax.ShapeDtypeStruct((grid * PROW, 2 * D), jnp.float32),
    )


_pack_u = _make_pack(N_USERS)
_pack_m = _make_pack(N_MOVIES)


def _gather_body(u_idx_hbm, m_idx_hbm, u2_hbm, m2_hbm,
                 u_bias_hbm, m_bias_hbm,
                 xu_out, xm_out, ub_out, mb_out,
                 uidx_v, midx_v, uq_v, mq_v, rows_v, ub_v, mb_v,
                 sem_e, sem_b):
    wid = lax.axis_index("s") * NC + lax.axis_index("c")
    base = wid * BPW
    prow0 = wid * NCH
    pltpu.sync_copy(u_idx_hbm.at[pl.ds(base, BPW)], uidx_v)
    pltpu.sync_copy(m_idx_hbm.at[pl.ds(base, BPW)], midx_v)
    bias_copies = []
    for j in range(NCH):
        sl = pl.ds(j * CH, CH)
        bias_copies.append(pltpu.async_copy(
            u_bias_hbm.at[uidx_v.at[sl]], ub_v.at[j], sem_b))
        bias_copies.append(pltpu.async_copy(
            m_bias_hbm.at[midx_v.at[sl]], mb_v.at[j], sem_b))
    for s in range(BPW // 16):
        sl = pl.ds(s * 16, 16)
        uv = uidx_v[sl]
        mv = midx_v[sl]
        uq_v[sl] = ((uv >> 13) << 12) + (uv & (PROW - 1))
        mq_v[sl] = ((mv >> 13) << 12) + (mv & (PROW - 1))
    for tab, qv, out in ((u2_hbm, uq_v, xu_out), (m2_hbm, mq_v, xm_out)):
        copies = []
        for j in range(NCH):
            sl = pl.ds(j * CH, CH)
            copies.append(pltpu.async_copy(
                tab.at[qv.at[sl]], rows_v.at[pl.ds(j * CH, CH)], sem_e))
        for c in copies:
            c.wait()
        pltpu.sync_copy(rows_v, out.at[pl.ds(base, BPW)])
    for c in bias_copies:
        c.wait()
    pltpu.sync_copy(ub_v, ub_out.at[pl.ds(prow0, NCH)])
    pltpu.sync_copy(mb_v, mb_out.at[pl.ds(prow0, NCH)])


_gather = pl.kernel(
    _gather_body,
    mesh=plsc.VectorSubcoreMesh(core_axis_name="c", subcore_axis_name="s"),
    out_type=[
        jax.ShapeDtypeStruct((B, 2 * D), jnp.float32),
        jax.ShapeDtypeStruct((B, 2 * D), jnp.float32),
        jax.ShapeDtypeStruct((B // CH, CH), jnp.float32),
        jax.ShapeDtypeStruct((B // CH, CH), jnp.float32),
    ],
    scratch_types=[
        pltpu.VMEM((BPW,), jnp.int32),
        pltpu.VMEM((BPW,), jnp.int32),
        pltpu.VMEM((BPW,), jnp.int32),
        pltpu.VMEM((BPW,), jnp.int32),
        pltpu.VMEM((BPW, 2 * D), jnp.float32),
        pltpu.VMEM((NCH, CH), jnp.float32),
        pltpu.VMEM((NCH, CH), jnp.float32),
        pltpu.SemaphoreType.DMA,
        pltpu.SemaphoreType.DMA,
    ],
    compiler_params=pltpu.CompilerParams(use_tc_tiling_on_sc=False),
)


BT = 1024
GRID = B // BT
PR = BT // CH


def _expand_col(pack_blk, P):
    row_ids = lax.broadcasted_iota(jnp.int32, (BT, PR), 0) // CH
    col_ids = lax.broadcasted_iota(jnp.int32, (BT, PR), 1)
    rowsel = (row_ids == col_ids).astype(jnp.float32)
    expanded = jnp.dot(rowsel, pack_blk, preferred_element_type=jnp.float32,
                       precision=P)
    lane = lax.broadcasted_iota(jnp.int32, (BT, CH), 1)
    bpos = lax.broadcasted_iota(jnp.int32, (BT, CH), 0) % CH
    return jnp.sum(jnp.where(lane == bpos, expanded, 0.0), axis=1,
                   keepdims=True)


def _mlp_body(xu_ref, xm_ref, up_ref, mp_ref, ub_ref, mb_ref,
              w1_ref, b1_ref, w2_ref, b2_ref, o_ref):
    P = lax.Precision.HIGHEST
    upar = _expand_col(up_ref[...], P)
    mpar = _expand_col(mp_ref[...], P)
    xu = xu_ref[...]
    xm = xm_ref[...]
    u = jnp.where(upar > 0.5, xu[:, D:2 * D], xu[:, 0:D])
    m = jnp.where(mpar > 0.5, xm[:, D:2 * D], xm[:, 0:D])
    w1 = w1_ref[...]
    acc = jnp.dot(u, w1[0:D, :], preferred_element_type=jnp.float32,
                  precision=P)
    acc = acc + jnp.dot(m, w1[D:2 * D, :], preferred_element_type=jnp.float32,
                        precision=P)
    acc = acc + jnp.dot(u * m, w1[2 * D:3 * D, :],
                        preferred_element_type=jnp.float32, precision=P)
    h = jnp.maximum(acc + b1_ref[...], 0.0)
    o = jnp.dot(h, w2_ref[...], preferred_element_type=jnp.float32,
                precision=P)
    bias_col = _expand_col(ub_ref[...] + mb_ref[...], P)
    o_ref[...] = jax.nn.sigmoid(o + b2_ref[0, 0] + bias_col)


_mlp = pl.pallas_call(
    _mlp_body,
    grid=(GRID,),
    in_specs=[
        pl.BlockSpec((BT, 2 * D), lambda i: (i, 0)),
        pl.BlockSpec((BT, 2 * D), lambda i: (i, 0)),
        pl.BlockSpec((PR, CH), lambda i: (i, 0)),
        pl.BlockSpec((PR, CH), lambda i: (i, 0)),
        pl.BlockSpec((PR, CH), lambda i: (i, 0)),
        pl.BlockSpec((PR, CH), lambda i: (i, 0)),
        pl.BlockSpec((3 * D, HIDDEN), lambda i: (0, 0)),
        pl.BlockSpec((1, HIDDEN), lambda i: (0, 0)),
        pl.BlockSpec((HIDDEN, 1), lambda i: (0, 0)),
        pl.BlockSpec((1, 1), lambda i: (0, 0)),
    ],
    out_specs=pl.BlockSpec((BT, 1), lambda i: (i, 0)),
    out_shape=jax.ShapeDtypeStruct((B, 1), jnp.float32),
)


def kernel(u_idx, m_idx, u_emb, m_emb, u_bias, m_bias, W1, b1, W2, b2):
    u_idx = u_idx.astype(jnp.int32)
    m_idx = m_idx.astype(jnp.int32)
    u2 = _pack_u(u_emb.T)
    m2 = _pack_m(m_emb.T)
    xu, xm, ub_g, mb_g = _gather(u_idx, m_idx, u2, m2,
                                 u_bias.reshape(-1), m_bias.reshape(-1))
    upar = ((u_idx >> 12) & 1).astype(jnp.float32).reshape(B // CH, CH)
    mpar = ((m_idx >> 12) & 1).astype(jnp.float32).reshape(B // CH, CH)
    out = _mlp(xu, xm, upar, mpar, ub_g, mb_g, W1, b1.reshape(1, HIDDEN),
               W2, b2.reshape(1, 1))
    return out.reshape(B)


# speedup vs baseline: 1.5617x; 1.5617x over previous
import jax
import jax.numpy as jnp
from jax import lax
from jax.experimental import pallas as pl
from jax.experimental.pallas import tpu as pltpu
from jax.experimental.pallas import tpu_sc as plsc

B = 16384
D = 64
HIDDEN = 128
N_USERS = 1000000
N_MOVIES = 100000
NC = 2
NS = 16
NW = NC * NS
BPW = B // NW
CH = 128
NCH = BPW // CH
PCOL = 8192
PROW = PCOL // 2


def _pack_body(t_ref, o_ref):
    v = t_ref[...]
    eye = (lax.broadcasted_iota(jnp.int32, (D, D), 0) ==
           lax.broadcasted_iota(jnp.int32, (D, D), 1)).astype(jnp.float32)
    dn = (((0,), (0,)), ((), ()))
    top = lax.dot_general(v[:, 0:PROW], eye, dn,
                          preferred_element_type=jnp.float32)
    bot = lax.dot_general(v[:, PROW:PCOL], eye, dn,
                          preferred_element_type=jnp.float32)
    o_ref[...] = jnp.concatenate([top, bot], axis=1)


def _make_pack(v_rows):
    grid = (v_rows + PCOL - 1) // PCOL
    return pl.pallas_call(
        _pack_body,
        grid=(grid,),
        in_specs=[pl.BlockSpec((D, PCOL), lambda i: (0, i))],
        out_specs=pl.BlockSpec((PROW, 2 * D), lambda i: (i, 0)),
        out_shape=jax.ShapeDtypeStruct((grid * PROW, 2 * D), jnp.float32),
        compiler_params=pltpu.CompilerParams(
            fuse_transposed_lhs_in_matmul=True),
    )


_pack_u = _make_pack(N_USERS)
_pack_m = _make_pack(N_MOVIES)


def _gather_body(u_idx_hbm, m_idx_hbm, u2_hbm, m2_hbm,
                 u_bias_hbm, m_bias_hbm,
                 xu_out, xm_out, ub_out, mb_out,
                 uidx_v, midx_v, uq_v, mq_v, rows_v, ub_v, mb_v,
                 sem_e, sem_b):
    wid = lax.axis_index("s") * NC + lax.axis_index("c")
    base = wid * BPW
    prow0 = wid * NCH
    pltpu.sync_copy(u_idx_hbm.at[pl.ds(base, BPW)], uidx_v)
    pltpu.sync_copy(m_idx_hbm.at[pl.ds(base, BPW)], midx_v)
    bias_copies = []
    for j in range(NCH):
        sl = pl.ds(j * CH, CH)
        bias_copies.append(pltpu.async_copy(
            u_bias_hbm.at[uidx_v.at[sl]], ub_v.at[j], sem_b))
        bias_copies.append(pltpu.async_copy(
            m_bias_hbm.at[midx_v.at[sl]], mb_v.at[j], sem_b))
    for s in range(BPW // 16):
        sl = pl.ds(s * 16, 16)
        uv = uidx_v[sl]
        mv = midx_v[sl]
        uq_v[sl] = ((uv >> 13) << 12) + (uv & (PROW - 1))
        mq_v[sl] = ((mv >> 13) << 12) + (mv & (PROW - 1))
    for tab, qv, out in ((u2_hbm, uq_v, xu_out), (m2_hbm, mq_v, xm_out)):
        copies = []
        for j in range(NCH):
            sl = pl.ds(j * CH, CH)
            copies.append(pltpu.async_copy(
                tab.at[qv.at[sl]], rows_v.at[pl.ds(j * CH, CH)], sem_e))
        for c in copies:
            c.wait()
        pltpu.sync_copy(rows_v, out.at[pl.ds(base, BPW)])
    for c in bias_copies:
        c.wait()
    pltpu.sync_copy(ub_v, ub_out.at[pl.ds(prow0, NCH)])
    pltpu.sync_copy(mb_v, mb_out.at[pl.ds(prow0, NCH)])


_gather = pl.kernel(
    _gather_body,
    mesh=plsc.VectorSubcoreMesh(core_axis_name="c", subcore_axis_name="s"),
    out_type=[
        jax.ShapeDtypeStruct((B, 2 * D), jnp.float32),
        jax.ShapeDtypeStruct((B, 2 * D), jnp.float32),
        jax.ShapeDtypeStruct((B // CH, CH), jnp.float32),
        jax.ShapeDtypeStruct((B // CH, CH), jnp.float32),
    ],
    scratch_types=[
        pltpu.VMEM((BPW,), jnp.int32),
        pltpu.VMEM((BPW,), jnp.int32),
        pltpu.VMEM((BPW,), jnp.int32),
        pltpu.VMEM((BPW,), jnp.int32),
        pltpu.VMEM((BPW, 2 * D), jnp.float32),
        pltpu.VMEM((NCH, CH), jnp.float32),
        pltpu.VMEM((NCH, CH), jnp.float32),
        pltpu.SemaphoreType.DMA,
        pltpu.SemaphoreType.DMA,
    ],
    compiler_params=pltpu.CompilerParams(use_tc_tiling_on_sc=False),
)


BT = 1024
GRID = B // BT
PR = BT // CH


def _expand_col(pack_blk, P):
    row_ids = lax.broadcasted_iota(jnp.int32, (BT, PR), 0) // CH
    col_ids = lax.broadcasted_iota(jnp.int32, (BT, PR), 1)
    rowsel = (row_ids == col_ids).astype(jnp.float32)
    expanded = jnp.dot(rowsel, pack_blk, preferred_element_type=jnp.float32,
                       precision=P)
    lane = lax.broadcasted_iota(jnp.int32, (BT, CH), 1)
    bpos = lax.broadcasted_iota(jnp.int32, (BT, CH), 0) % CH
    return jnp.sum(jnp.where(lane == bpos, expanded, 0.0), axis=1,
                   keepdims=True)


def _mlp_body(xu_ref, xm_ref, up_ref, mp_ref, ub_ref, mb_ref,
              w1_ref, b1_ref, w2_ref, b2_ref, o_ref):
    P = lax.Precision.HIGHEST
    upar = _expand_col(up_ref[...], P)
    mpar = _expand_col(mp_ref[...], P)
    xu = xu_ref[...]
    xm = xm_ref[...]
    u = jnp.where(upar > 0.5, xu[:, D:2 * D], xu[:, 0:D])
    m = jnp.where(mpar > 0.5, xm[:, D:2 * D], xm[:, 0:D])
    w1 = w1_ref[...]
    acc = jnp.dot(u, w1[0:D, :], preferred_element_type=jnp.float32,
                  precision=P)
    acc = acc + jnp.dot(m, w1[D:2 * D, :], preferred_element_type=jnp.float32,
                        precision=P)
    acc = acc + jnp.dot(u * m, w1[2 * D:3 * D, :],
                        preferred_element_type=jnp.float32, precision=P)
    h = jnp.maximum(acc + b1_ref[...], 0.0)
    o = jnp.dot(h, w2_ref[...], preferred_element_type=jnp.float32,
                precision=P)
    bias_col = _expand_col(ub_ref[...] + mb_ref[...], P)
    o_ref[...] = jax.nn.sigmoid(o + b2_ref[0, 0] + bias_col)


_mlp = pl.pallas_call(
    _mlp_body,
    grid=(GRID,),
    in_specs=[
        pl.BlockSpec((BT, 2 * D), lambda i: (i, 0)),
        pl.BlockSpec((BT, 2 * D), lambda i: (i, 0)),
        pl.BlockSpec((PR, CH), lambda i: (i, 0)),
        pl.BlockSpec((PR, CH), lambda i: (i, 0)),
        pl.BlockSpec((PR, CH), lambda i: (i, 0)),
        pl.BlockSpec((PR, CH), lambda i: (i, 0)),
        pl.BlockSpec((3 * D, HIDDEN), lambda i: (0, 0)),
        pl.BlockSpec((1, HIDDEN), lambda i: (0, 0)),
        pl.BlockSpec((HIDDEN, 1), lambda i: (0, 0)),
        pl.BlockSpec((1, 1), lambda i: (0, 0)),
    ],
    out_specs=pl.BlockSpec((BT, 1), lambda i: (i, 0)),
    out_shape=jax.ShapeDtypeStruct((B, 1), jnp.float32),
)


def kernel(u_idx, m_idx, u_emb, m_emb, u_bias, m_bias, W1, b1, W2, b2):
    u_idx = u_idx.astype(jnp.int32)
    m_idx = m_idx.astype(jnp.int32)
    u2 = _pack_u(u_emb.T)
    m2 = _pack_m(m_emb.T)
    xu, xm, ub_g, mb_g = _gather(u_idx, m_idx, u2, m2,
                                 u_bias.reshape(-1), m_bias.reshape(-1))
    upar = ((u_idx >> 12) & 1).astype(jnp.float32).reshape(B // CH, CH)
    mpar = ((m_idx >> 12) & 1).astype(jnp.float32).reshape(B // CH, CH)
    out = _mlp(xu, xm, upar, mpar, ub_g, mb_g, W1, b1.reshape(1, HIDDEN),
               W2, b2.reshape(1, 1))
    return out.reshape(B)


# speedup vs baseline: 2.0641x; 1.3217x over previous
import jax
import jax.numpy as jnp
from jax import lax
from jax.experimental import pallas as pl
from jax.experimental.pallas import tpu as pltpu
from jax.experimental.pallas import tpu_sc as plsc

B = 16384
D = 64
HIDDEN = 128
N_USERS = 1000000
N_MOVIES = 100000
NC = 2
NS = 16
NW = NC * NS
BPW = B // NW
CH = 128
NCH = BPW // CH
PCOL = 16384
PROW = PCOL // 2


def _pack_body(t_ref, o_ref):
    v = t_ref[...]
    eye = (lax.broadcasted_iota(jnp.int32, (D, D), 0) ==
           lax.broadcasted_iota(jnp.int32, (D, D), 1)).astype(jnp.float32)
    dn = (((0,), (0,)), ((), ()))
    top = lax.dot_general(v[:, 0:PROW], eye, dn,
                          preferred_element_type=jnp.float32)
    bot = lax.dot_general(v[:, PROW:PCOL], eye, dn,
                          preferred_element_type=jnp.float32)
    o_ref[...] = jnp.concatenate([top, bot], axis=1)


def _make_pack(v_rows):
    grid = (v_rows + PCOL - 1) // PCOL
    return pl.pallas_call(
        _pack_body,
        grid=(grid,),
        in_specs=[pl.BlockSpec((D, PCOL), lambda i: (0, i))],
        out_specs=pl.BlockSpec((PROW, 2 * D), lambda i: (i, 0)),
        out_shape=jax.ShapeDtypeStruct((grid * PROW, 2 * D), jnp.float32),
        compiler_params=pltpu.CompilerParams(
            fuse_transposed_lhs_in_matmul=True),
    )


_pack_u = _make_pack(N_USERS)
_pack_m = _make_pack(N_MOVIES)


def _gather_body(u_idx_hbm, m_idx_hbm, u2_hbm, m2_hbm,
                 u_bias_hbm, m_bias_hbm,
                 xu_out, xm_out, ub_out, mb_out,
                 uidx_v, midx_v, uq_v, mq_v, rows_v, ub_v, mb_v,
                 sem_e, sem_b):
    wid = lax.axis_index("s") * NC + lax.axis_index("c")
    base = wid * BPW
    prow0 = wid * NCH
    pltpu.sync_copy(u_idx_hbm.at[pl.ds(base, BPW)], uidx_v)
    pltpu.sync_copy(m_idx_hbm.at[pl.ds(base, BPW)], midx_v)
    bias_copies = []
    for j in range(NCH):
        sl = pl.ds(j * CH, CH)
        bias_copies.append(pltpu.async_copy(
            u_bias_hbm.at[uidx_v.at[sl]], ub_v.at[j], sem_b))
        bias_copies.append(pltpu.async_copy(
            m_bias_hbm.at[midx_v.at[sl]], mb_v.at[j], sem_b))
    for s in range(BPW // 16):
        sl = pl.ds(s * 16, 16)
        uv = uidx_v[sl]
        mv = midx_v[sl]
        uq_v[sl] = ((uv >> 14) << 13) + (uv & (PROW - 1))
        mq_v[sl] = ((mv >> 14) << 13) + (mv & (PROW - 1))
    for tab, qv, out in ((u2_hbm, uq_v, xu_out), (m2_hbm, mq_v, xm_out)):
        copies = []
        for j in range(NCH):
            sl = pl.ds(j * CH, CH)
            copies.append(pltpu.async_copy(
                tab.at[qv.at[sl]], rows_v.at[pl.ds(j * CH, CH)], sem_e))
        for c in copies:
            c.wait()
        pltpu.sync_copy(rows_v, out.at[pl.ds(base, BPW)])
    for c in bias_copies:
        c.wait()
    pltpu.sync_copy(ub_v, ub_out.at[pl.ds(prow0, NCH)])
    pltpu.sync_copy(mb_v, mb_out.at[pl.ds(prow0, NCH)])


_gather = pl.kernel(
    _gather_body,
    mesh=plsc.VectorSubcoreMesh(core_axis_name="c", subcore_axis_name="s"),
    out_type=[
        jax.ShapeDtypeStruct((B, 2 * D), jnp.float32),
        jax.ShapeDtypeStruct((B, 2 * D), jnp.float32),
        jax.ShapeDtypeStruct((B // CH, CH), jnp.float32),
        jax.ShapeDtypeStruct((B // CH, CH), jnp.float32),
    ],
    scratch_types=[
        pltpu.VMEM((BPW,), jnp.int32),
        pltpu.VMEM((BPW,), jnp.int32),
        pltpu.VMEM((BPW,), jnp.int32),
        pltpu.VMEM((BPW,), jnp.int32),
        pltpu.VMEM((BPW, 2 * D), jnp.float32),
        pltpu.VMEM((NCH, CH), jnp.float32),
        pltpu.VMEM((NCH, CH), jnp.float32),
        pltpu.SemaphoreType.DMA,
        pltpu.SemaphoreType.DMA,
    ],
    compiler_params=pltpu.CompilerParams(use_tc_tiling_on_sc=False),
)


BT = 1024
GRID = B // BT
PR = BT // CH


def _expand_col(pack_blk, P):
    row_ids = lax.broadcasted_iota(jnp.int32, (BT, PR), 0) // CH
    col_ids = lax.broadcasted_iota(jnp.int32, (BT, PR), 1)
    rowsel = (row_ids == col_ids).astype(jnp.float32)
    expanded = jnp.dot(rowsel, pack_blk, preferred_element_type=jnp.float32,
                       precision=P)
    lane = lax.broadcasted_iota(jnp.int32, (BT, CH), 1)
    bpos = lax.broadcasted_iota(jnp.int32, (BT, CH), 0) % CH
    return jnp.sum(jnp.where(lane == bpos, expanded, 0.0), axis=1,
                   keepdims=True)


def _mlp_body(xu_ref, xm_ref, up_ref, mp_ref, ub_ref, mb_ref,
              w1_ref, b1_ref, w2_ref, b2_ref, o_ref):
    P = None
    upar = _expand_col(up_ref[...], P)
    mpar = _expand_col(mp_ref[...], P)
    xu = xu_ref[...]
    xm = xm_ref[...]
    u = jnp.where(upar > 0.5, xu[:, D:2 * D], xu[:, 0:D])
    m = jnp.where(mpar > 0.5, xm[:, D:2 * D], xm[:, 0:D])
    w1 = w1_ref[...]
    acc = jnp.dot(u, w1[0:D, :], preferred_element_type=jnp.float32,
                  precision=P)
    acc = acc + jnp.dot(m, w1[D:2 * D, :], preferred_element_type=jnp.float32,
                        precision=P)
    acc = acc + jnp.dot(u * m, w1[2 * D:3 * D, :],
                        preferred_element_type=jnp.float32, precision=P)
    h = jnp.maximum(acc + b1_ref[...], 0.0)
    o = jnp.dot(h, w2_ref[...], preferred_element_type=jnp.float32,
                precision=P)
    bias_col = _expand_col(ub_ref[...] + mb_ref[...], P)
    o_ref[...] = jax.nn.sigmoid(o + b2_ref[0, 0] + bias_col)


_mlp = pl.pallas_call(
    _mlp_body,
    grid=(GRID,),
    in_specs=[
        pl.BlockSpec((BT, 2 * D), lambda i: (i, 0)),
        pl.BlockSpec((BT, 2 * D), lambda i: (i, 0)),
        pl.BlockSpec((PR, CH), lambda i: (i, 0)),
        pl.BlockSpec((PR, CH), lambda i: (i, 0)),
        pl.BlockSpec((PR, CH), lambda i: (i, 0)),
        pl.BlockSpec((PR, CH), lambda i: (i, 0)),
        pl.BlockSpec((3 * D, HIDDEN), lambda i: (0, 0)),
        pl.BlockSpec((1, HIDDEN), lambda i: (0, 0)),
        pl.BlockSpec((HIDDEN, 1), lambda i: (0, 0)),
        pl.BlockSpec((1, 1), lambda i: (0, 0)),
    ],
    out_specs=pl.BlockSpec((BT, 1), lambda i: (i, 0)),
    out_shape=jax.ShapeDtypeStruct((B, 1), jnp.float32),
)


def kernel(u_idx, m_idx, u_emb, m_emb, u_bias, m_bias, W1, b1, W2, b2):
    u_idx = u_idx.astype(jnp.int32)
    m_idx = m_idx.astype(jnp.int32)
    u2 = _pack_u(u_emb.T)
    m2 = _pack_m(m_emb.T)
    xu, xm, ub_g, mb_g = _gather(u_idx, m_idx, u2, m2,
                                 u_bias.reshape(-1), m_bias.reshape(-1))
    upar = ((u_idx >> 13) & 1).astype(jnp.float32).reshape(B // CH, CH)
    mpar = ((m_idx >> 13) & 1).astype(jnp.float32).reshape(B // CH, CH)
    out = _mlp(xu, xm, upar, mpar, ub_g, mb_g, W1, b1.reshape(1, HIDDEN),
               W2, b2.reshape(1, 1))
    return out.reshape(B)


# speedup vs baseline: 2.1382x; 1.0359x over previous
import jax
import jax.numpy as jnp
from jax import lax
from jax.experimental import pallas as pl
from jax.experimental.pallas import tpu as pltpu
from jax.experimental.pallas import tpu_sc as plsc

B = 16384
D = 64
HIDDEN = 128
N_USERS = 1000000
N_MOVIES = 100000
NC = 2
NS = 16
NW = NC * NS
BPW = B // NW
CH = 128
NCH = BPW // CH
PCOL = 32768
PROW = PCOL // 2


def _pack_body(t_ref, o_ref):
    v = t_ref[...]
    eye = (lax.broadcasted_iota(jnp.int32, (D, D), 0) ==
           lax.broadcasted_iota(jnp.int32, (D, D), 1)).astype(jnp.float32)
    dn = (((0,), (0,)), ((), ()))
    top = lax.dot_general(v[:, 0:PROW], eye, dn,
                          preferred_element_type=jnp.float32)
    bot = lax.dot_general(v[:, PROW:PCOL], eye, dn,
                          preferred_element_type=jnp.float32)
    o_ref[...] = jnp.concatenate([top, bot], axis=1)


def _make_pack(v_rows):
    grid = (v_rows + PCOL - 1) // PCOL
    return pl.pallas_call(
        _pack_body,
        grid=(grid,),
        in_specs=[pl.BlockSpec((D, PCOL), lambda i: (0, i))],
        out_specs=pl.BlockSpec((PROW, 2 * D), lambda i: (i, 0)),
        out_shape=jax.ShapeDtypeStruct((grid * PROW, 2 * D), jnp.float32),
        compiler_params=pltpu.CompilerParams(
            fuse_transposed_lhs_in_matmul=True),
    )


_pack_u = _make_pack(N_USERS)
_pack_m = _make_pack(N_MOVIES)


def _gather_body(u_idx_hbm, m_idx_hbm, u2_hbm, m2_hbm,
                 u_bias_hbm, m_bias_hbm,
                 xu_out, xm_out, ub_out, mb_out,
                 uidx_v, midx_v, uq_v, mq_v, rows_v, ub_v, mb_v,
                 sem_e, sem_b):
    wid = lax.axis_index("s") * NC + lax.axis_index("c")
    base = wid * BPW
    prow0 = wid * NCH
    pltpu.sync_copy(u_idx_hbm.at[pl.ds(base, BPW)], uidx_v)
    pltpu.sync_copy(m_idx_hbm.at[pl.ds(base, BPW)], midx_v)
    bias_copies = []
    for j in range(NCH):
        sl = pl.ds(j * CH, CH)
        bias_copies.append(pltpu.async_copy(
            u_bias_hbm.at[uidx_v.at[sl]], ub_v.at[j], sem_b))
        bias_copies.append(pltpu.async_copy(
            m_bias_hbm.at[midx_v.at[sl]], mb_v.at[j], sem_b))
    for s in range(BPW // 16):
        sl = pl.ds(s * 16, 16)
        uv = uidx_v[sl]
        mv = midx_v[sl]
        uq_v[sl] = ((uv >> 15) << 14) + (uv & (PROW - 1))
        mq_v[sl] = ((mv >> 15) << 14) + (mv & (PROW - 1))
    for tab, qv, out in ((u2_hbm, uq_v, xu_out), (m2_hbm, mq_v, xm_out)):
        copies = []
        for j in range(NCH):
            sl = pl.ds(j * CH, CH)
            copies.append(pltpu.async_copy(
                tab.at[qv.at[sl]], rows_v.at[pl.ds(j * CH, CH)], sem_e))
        for c in copies:
            c.wait()
        pltpu.sync_copy(rows_v, out.at[pl.ds(base, BPW)])
    for c in bias_copies:
        c.wait()
    pltpu.sync_copy(ub_v, ub_out.at[pl.ds(prow0, NCH)])
    pltpu.sync_copy(mb_v, mb_out.at[pl.ds(prow0, NCH)])


_gather = pl.kernel(
    _gather_body,
    mesh=plsc.VectorSubcoreMesh(core_axis_name="c", subcore_axis_name="s"),
    out_type=[
        jax.ShapeDtypeStruct((B, 2 * D), jnp.float32),
        jax.ShapeDtypeStruct((B, 2 * D), jnp.float32),
        jax.ShapeDtypeStruct((B // CH, CH), jnp.float32),
        jax.ShapeDtypeStruct((B // CH, CH), jnp.float32),
    ],
    scratch_types=[
        pltpu.VMEM((BPW,), jnp.int32),
        pltpu.VMEM((BPW,), jnp.int32),
        pltpu.VMEM((BPW,), jnp.int32),
        pltpu.VMEM((BPW,), jnp.int32),
        pltpu.VMEM((BPW, 2 * D), jnp.float32),
        pltpu.VMEM((NCH, CH), jnp.float32),
        pltpu.VMEM((NCH, CH), jnp.float32),
        pltpu.SemaphoreType.DMA,
        pltpu.SemaphoreType.DMA,
    ],
    compiler_params=pltpu.CompilerParams(use_tc_tiling_on_sc=False),
)


BT = 2048
GRID = B // BT
PR = BT // CH


def _expand_col(pack_blk, P):
    row_ids = lax.broadcasted_iota(jnp.int32, (BT, PR), 0) // CH
    col_ids = lax.broadcasted_iota(jnp.int32, (BT, PR), 1)
    rowsel = (row_ids == col_ids).astype(jnp.float32)
    expanded = jnp.dot(rowsel, pack_blk, preferred_element_type=jnp.float32,
                       precision=P)
    lane = lax.broadcasted_iota(jnp.int32, (BT, CH), 1)
    bpos = lax.broadcasted_iota(jnp.int32, (BT, CH), 0) % CH
    return jnp.sum(jnp.where(lane == bpos, expanded, 0.0), axis=1,
                   keepdims=True)


def _mlp_body(xu_ref, xm_ref, up_ref, mp_ref, ub_ref, mb_ref,
              w1_ref, b1_ref, w2_ref, b2_ref, o_ref):
    P = None
    upar = _expand_col(up_ref[...], P)
    mpar = _expand_col(mp_ref[...], P)
    xu = xu_ref[...]
    xm = xm_ref[...]
    u = jnp.where(upar > 0.5, xu[:, D:2 * D], xu[:, 0:D])
    m = jnp.where(mpar > 0.5, xm[:, D:2 * D], xm[:, 0:D])
    w1 = w1_ref[...]
    acc = jnp.dot(u, w1[0:D, :], preferred_element_type=jnp.float32,
                  precision=P)
    acc = acc + jnp.dot(m, w1[D:2 * D, :], preferred_element_type=jnp.float32,
                        precision=P)
    acc = acc + jnp.dot(u * m, w1[2 * D:3 * D, :],
                        preferred_element_type=jnp.float32, precision=P)
    h = jnp.maximum(acc + b1_ref[...], 0.0)
    o = jnp.dot(h, w2_ref[...], preferred_element_type=jnp.float32,
                precision=P)
    bias_col = _expand_col(ub_ref[...] + mb_ref[...], P)
    o_ref[...] = jax.nn.sigmoid(o + b2_ref[0, 0] + bias_col)


_mlp = pl.pallas_call(
    _mlp_body,
    grid=(GRID,),
    in_specs=[
        pl.BlockSpec((BT, 2 * D), lambda i: (i, 0)),
        pl.BlockSpec((BT, 2 * D), lambda i: (i, 0)),
        pl.BlockSpec((PR, CH), lambda i: (i, 0)),
        pl.BlockSpec((PR, CH), lambda i: (i, 0)),
        pl.BlockSpec((PR, CH), lambda i: (i, 0)),
        pl.BlockSpec((PR, CH), lambda i: (i, 0)),
        pl.BlockSpec((3 * D, HIDDEN), lambda i: (0, 0)),
        pl.BlockSpec((1, HIDDEN), lambda i: (0, 0)),
        pl.BlockSpec((HIDDEN, 1), lambda i: (0, 0)),
        pl.BlockSpec((1, 1), lambda i: (0, 0)),
    ],
    out_specs=pl.BlockSpec((BT, 1), lambda i: (i, 0)),
    out_shape=jax.ShapeDtypeStruct((B, 1), jnp.float32),
)


def kernel(u_idx, m_idx, u_emb, m_emb, u_bias, m_bias, W1, b1, W2, b2):
    u_idx = u_idx.astype(jnp.int32)
    m_idx = m_idx.astype(jnp.int32)
    u2 = _pack_u(u_emb.T)
    m2 = _pack_m(m_emb.T)
    xu, xm, ub_g, mb_g = _gather(u_idx, m_idx, u2, m2,
                                 u_bias.reshape(-1), m_bias.reshape(-1))
    upar = ((u_idx >> 14) & 1).astype(jnp.float32).reshape(B // CH, CH)
    mpar = ((m_idx >> 14) & 1).astype(jnp.float32).reshape(B // CH, CH)
    out = _mlp(xu, xm, upar, mpar, ub_g, mb_g, W1, b1.reshape(1, HIDDEN),
               W2, b2.reshape(1, 1))
    return out.reshape(B)
